# Initial kernel scaffold; baseline (speedup 1.0000x reference)
import jax
import jax.numpy as jnp
from jax.experimental import pallas as pl


def kernel(ego_emb, neighbor_emb, relation_ids, rel_table, W1_w, W1_b, W2_w, W2_b):
    raise NotImplementedError("write your pallas kernel here")


# speedup vs baseline: 2.6677x; 2.6677x over previous
import jax
import jax.numpy as jnp
from jax.experimental import pallas as pl
from jax.experimental.pallas import tpu as pltpu

_E = 160000
_D = 256
_R = 64
_BE = 1600
_NB = _E // _BE


def _relproj_kernel(rel_table_ref, w_rel_ref, out_ref):
    out_ref[...] = jnp.dot(rel_table_ref[...], w_rel_ref[...],
                           preferred_element_type=jnp.float32)


def _main_kernel(ids_ref, ego_ref, nbr_ref, w_ego_ref, w_nbr_ref,
                 rel_proj_ref, b1_ref, w2_ref, b2_ref, tt_ref, attn_ref):
    ego = ego_ref[...]
    nbr = nbr_ref[...]
    ids = ids_ref[0, 0, :]
    onehot = (ids[:, None]
              == jax.lax.broadcasted_iota(jnp.int32, (1, _R), 1)
              ).astype(jnp.float32)
    tt = (jnp.dot(ego, w_ego_ref[...], preferred_element_type=jnp.float32)
          + jnp.dot(nbr, w_nbr_ref[...], preferred_element_type=jnp.float32)
          + jnp.dot(onehot, rel_proj_ref[...],
                    preferred_element_type=jnp.float32)
          + b1_ref[...])
    tt_ref[...] = tt
    a = jnp.sum(tt * w2_ref[...], axis=1, keepdims=True) + b2_ref[0, 0]
    attn_ref[...] = jnp.where(a >= 0, a, 0.2 * a)


def kernel(ego_emb, neighbor_emb, relation_ids, rel_table, W1_w, W1_b,
           W2_w, W2_b):
    wt = W1_w.T
    w_ego = wt[0:_D]
    w_rel = wt[_D:2 * _D]
    w_nbr = wt[2 * _D:3 * _D]

    rel_proj = pl.pallas_call(
        _relproj_kernel,
        out_shape=jax.ShapeDtypeStruct((_R, _D), jnp.float32),
    )(rel_table, w_rel)

    ids3 = relation_ids.astype(jnp.int32).reshape(_NB, 1, _BE)
    b1 = W1_b.reshape(1, _D)
    w2 = W2_w.reshape(1, _D)
    b2 = W2_b.reshape(1, 1)

    tt, attn = pl.pallas_call(
        _main_kernel,
        grid=(_NB,),
        in_specs=[
            pl.BlockSpec((1, 1, _BE), lambda i: (i, 0, 0)),
            pl.BlockSpec((_BE, _D), lambda i: (i, 0)),
            pl.BlockSpec((_BE, _D), lambda i: (i, 0)),
            pl.BlockSpec((_D, _D), lambda i: (0, 0)),
            pl.BlockSpec((_D, _D), lambda i: (0, 0)),
            pl.BlockSpec((_R, _D), lambda i: (0, 0)),
            pl.BlockSpec((1, _D), lambda i: (0, 0)),
            pl.BlockSpec((1, _D), lambda i: (0, 0)),
            pl.BlockSpec((1, 1), lambda i: (0, 0)),
        ],
        out_specs=[
            pl.BlockSpec((_BE, _D), lambda i: (i, 0)),
            pl.BlockSpec((_BE, 1), lambda i: (i, 0)),
        ],
        out_shape=[
            jax.ShapeDtypeStruct((_E, _D), jnp.float32),
            jax.ShapeDtypeStruct((_E, 1), jnp.float32),
        ],
        compiler_params=pltpu.CompilerParams(
            dimension_semantics=("parallel",)),
    )(ids3, ego_emb, neighbor_emb, w_ego, w_nbr, rel_proj, b1, w2, b2)
    return (tt, attn)


# speedup vs baseline: 2.9331x; 1.0995x over previous
import jax
import jax.numpy as jnp
from jax.experimental import pallas as pl
from jax.experimental.pallas import tpu as pltpu

_E = 160000
_D = 256
_R = 64
_BE = 3200
_NB = _E // _BE


def _relproj_kernel(rel_table_ref, w_rel_ref, out_ref):
    out_ref[...] = jnp.dot(rel_table_ref[...], w_rel_ref[...],
                           preferred_element_type=jnp.float32)


def _main_kernel(ids_ref, ego_ref, nbr_ref, w_ego_ref, w_nbr_ref,
                 rel_proj_ref, b1_ref, w2_ref, b2_ref, tt_ref, attn_ref):
    ego = ego_ref[...]
    nbr = nbr_ref[...]
    ids = ids_ref[0, 0, :]
    onehot = (ids[:, None]
              == jax.lax.broadcasted_iota(jnp.int32, (1, _R), 1)
              ).astype(jnp.float32)
    tt = (jnp.dot(ego, w_ego_ref[...], preferred_element_type=jnp.float32)
          + jnp.dot(nbr, w_nbr_ref[...], preferred_element_type=jnp.float32)
          + jnp.dot(onehot, rel_proj_ref[...],
                    preferred_element_type=jnp.float32)
          + b1_ref[...])
    tt_ref[...] = tt
    a = jnp.sum(tt * w2_ref[...], axis=1, keepdims=True) + b2_ref[0, 0]
    attn_ref[...] = jnp.where(a >= 0, a, 0.2 * a)


def kernel(ego_emb, neighbor_emb, relation_ids, rel_table, W1_w, W1_b,
           W2_w, W2_b):
    wt = W1_w.T
    w_ego = wt[0:_D]
    w_rel = wt[_D:2 * _D]
    w_nbr = wt[2 * _D:3 * _D]

    rel_proj = pl.pallas_call(
        _relproj_kernel,
        out_shape=jax.ShapeDtypeStruct((_R, _D), jnp.float32),
    )(rel_table, w_rel)

    ids3 = relation_ids.astype(jnp.int32).reshape(_NB, 1, _BE)
    b1 = W1_b.reshape(1, _D)
    w2 = W2_w.reshape(1, _D)
    b2 = W2_b.reshape(1, 1)

    tt, attn = pl.pallas_call(
        _main_kernel,
        grid=(_NB,),
        in_specs=[
            pl.BlockSpec((1, 1, _BE), lambda i: (i, 0, 0)),
            pl.BlockSpec((_BE, _D), lambda i: (i, 0)),
            pl.BlockSpec((_BE, _D), lambda i: (i, 0)),
            pl.BlockSpec((_D, _D), lambda i: (0, 0)),
            pl.BlockSpec((_D, _D), lambda i: (0, 0)),
            pl.BlockSpec((_R, _D), lambda i: (0, 0)),
            pl.BlockSpec((1, _D), lambda i: (0, 0)),
            pl.BlockSpec((1, _D), lambda i: (0, 0)),
            pl.BlockSpec((1, 1), lambda i: (0, 0)),
        ],
        out_specs=[
            pl.BlockSpec((_BE, _D), lambda i: (i, 0)),
            pl.BlockSpec((_BE, 1), lambda i: (i, 0)),
        ],
        out_shape=[
            jax.ShapeDtypeStruct((_E, _D), jnp.float32),
            jax.ShapeDtypeStruct((_E, 1), jnp.float32),
        ],
        compiler_params=pltpu.CompilerParams(
            dimension_semantics=("parallel",)),
    )(ids3, ego_emb, neighbor_emb, w_ego, w_nbr, rel_proj, b1, w2, b2)
    return (tt, attn)


# speedup vs baseline: 2.9660x; 1.0112x over previous
import jax
import jax.numpy as jnp
from jax.experimental import pallas as pl
from jax.experimental.pallas import tpu as pltpu

_E = 160000
_D = 256
_R = 64
_BE = 6400
_NB = _E // _BE


def _relproj_kernel(rel_table_ref, w_rel_ref, out_ref):
    out_ref[...] = jnp.dot(rel_table_ref[...], w_rel_ref[...],
                           preferred_element_type=jnp.float32)


def _main_kernel(ids_ref, ego_ref, nbr_ref, w_ego_ref, w_nbr_ref,
                 rel_proj_ref, b1_ref, w2_ref, b2_ref, tt_ref, attn_ref):
    ego = ego_ref[...]
    nbr = nbr_ref[...]
    ids = ids_ref[0, 0, :]
    onehot = (ids[:, None]
              == jax.lax.broadcasted_iota(jnp.int32, (1, _R), 1)
              ).astype(jnp.float32)
    tt = (jnp.dot(ego, w_ego_ref[...], preferred_element_type=jnp.float32)
          + jnp.dot(nbr, w_nbr_ref[...], preferred_element_type=jnp.float32)
          + jnp.dot(onehot, rel_proj_ref[...],
                    preferred_element_type=jnp.float32)
          + b1_ref[...])
    tt_ref[...] = tt
    a = jnp.sum(tt * w2_ref[...], axis=1, keepdims=True) + b2_ref[0, 0]
    attn_ref[...] = jnp.where(a >= 0, a, 0.2 * a)


def kernel(ego_emb, neighbor_emb, relation_ids, rel_table, W1_w, W1_b,
           W2_w, W2_b):
    wt = W1_w.T
    w_ego = wt[0:_D]
    w_rel = wt[_D:2 * _D]
    w_nbr = wt[2 * _D:3 * _D]

    rel_proj = pl.pallas_call(
        _relproj_kernel,
        out_shape=jax.ShapeDtypeStruct((_R, _D), jnp.float32),
    )(rel_table, w_rel)

    ids3 = relation_ids.astype(jnp.int32).reshape(_NB, 1, _BE)
    b1 = W1_b.reshape(1, _D)
    w2 = W2_w.reshape(1, _D)
    b2 = W2_b.reshape(1, 1)

    tt, attn = pl.pallas_call(
        _main_kernel,
        grid=(_NB,),
        in_specs=[
            pl.BlockSpec((1, 1, _BE), lambda i: (i, 0, 0)),
            pl.BlockSpec((_BE, _D), lambda i: (i, 0)),
            pl.BlockSpec((_BE, _D), lambda i: (i, 0)),
            pl.BlockSpec((_D, _D), lambda i: (0, 0)),
            pl.BlockSpec((_D, _D), lambda i: (0, 0)),
            pl.BlockSpec((_R, _D), lambda i: (0, 0)),
            pl.BlockSpec((1, _D), lambda i: (0, 0)),
            pl.BlockSpec((1, _D), lambda i: (0, 0)),
            pl.BlockSpec((1, 1), lambda i: (0, 0)),
        ],
        out_specs=[
            pl.BlockSpec((_BE, _D), lambda i: (i, 0)),
            pl.BlockSpec((_BE, 1), lambda i: (i, 0)),
        ],
        out_shape=[
            jax.ShapeDtypeStruct((_E, _D), jnp.float32),
            jax.ShapeDtypeStruct((_E, 1), jnp.float32),
        ],
        compiler_params=pltpu.CompilerParams(
            dimension_semantics=("parallel",)),
    )(ids3, ego_emb, neighbor_emb, w_ego, w_nbr, rel_proj, b1, w2, b2)
    return (tt, attn)


# speedup vs baseline: 3.0046x; 1.0130x over previous
import jax
import jax.numpy as jnp
from jax.experimental import pallas as pl
from jax.experimental.pallas import tpu as pltpu

_E = 160000
_D = 256
_R = 64
_BE = 6400
_NB = _E // _BE

_DN_ACT = (((1,), (1,)), ((), ()))
_DN_OH = (((1,), (0,)), ((), ()))


def _main_kernel(ids_ref, ego_ref, nbr_ref, w1_ref, rel_table_ref,
                 b1_ref, w2_ref, b2_ref, tt_ref, attn_ref):
    w1 = w1_ref[...]
    rel_proj = jax.lax.dot_general(
        rel_table_ref[...], w1[:, _D:2 * _D], _DN_ACT,
        preferred_element_type=jnp.float32)
    ids = ids_ref[0, 0, :]
    onehot = (ids[:, None]
              == jax.lax.broadcasted_iota(jnp.int32, (1, _R), 1)
              ).astype(jnp.float32)
    tt = (jax.lax.dot_general(ego_ref[...], w1[:, 0:_D], _DN_ACT,
                              preferred_element_type=jnp.float32)
          + jax.lax.dot_general(nbr_ref[...], w1[:, 2 * _D:3 * _D], _DN_ACT,
                                preferred_element_type=jnp.float32)
          + jax.lax.dot_general(onehot, rel_proj, _DN_OH,
                                preferred_element_type=jnp.float32)
          + b1_ref[...])
    tt_ref[...] = tt
    a = jnp.sum(tt * w2_ref[...], axis=1, keepdims=True) + b2_ref[0, 0]
    attn_ref[...] = jnp.where(a >= 0, a, 0.2 * a)


def kernel(ego_emb, neighbor_emb, relation_ids, rel_table, W1_w, W1_b,
           W2_w, W2_b):
    ids3 = relation_ids.astype(jnp.int32).reshape(_NB, 1, _BE)
    b1 = W1_b.reshape(1, _D)
    w2 = W2_w.reshape(1, _D)
    b2 = W2_b.reshape(1, 1)

    tt, attn = pl.pallas_call(
        _main_kernel,
        grid=(_NB,),
        in_specs=[
            pl.BlockSpec((1, 1, _BE), lambda i: (i, 0, 0)),
            pl.BlockSpec((_BE, _D), lambda i: (i, 0)),
            pl.BlockSpec((_BE, _D), lambda i: (i, 0)),
            pl.BlockSpec((_D, 3 * _D), lambda i: (0, 0)),
            pl.BlockSpec((_R, _D), lambda i: (0, 0)),
            pl.BlockSpec((1, _D), lambda i: (0, 0)),
            pl.BlockSpec((1, _D), lambda i: (0, 0)),
            pl.BlockSpec((1, 1), lambda i: (0, 0)),
        ],
        out_specs=[
            pl.BlockSpec((_BE, _D), lambda i: (i, 0)),
            pl.BlockSpec((_BE, 1), lambda i: (i, 0)),
        ],
        out_shape=[
            jax.ShapeDtypeStruct((_E, _D), jnp.float32),
            jax.ShapeDtypeStruct((_E, 1), jnp.float32),
        ],
        compiler_params=pltpu.CompilerParams(
            dimension_semantics=("parallel",)),
    )(ids3, ego_emb, neighbor_emb, W1_w, rel_table, b1, w2, b2)
    return (tt, attn)
